# baseline (speedup 1.0000x reference)
import functools

import jax
import jax.numpy as jnp
from jax import lax
from jax.experimental import pallas as pl
from jax.experimental.pallas import tpu as pltpu
from jax.experimental.pallas import tpu_sc as plsc

N = 1000000
B = 16384
NC = 2
NS = 16
NW = NC * NS
BPW = B // NW
CHUNK = 64
NCH = BPW // CHUNK
VPAD = 1000064
NT = VPAD // 128


def _worker_id_and_idx(tile_hbm, lane_hbm, tile_v, lane_v):
    c = lax.axis_index("c")
    s = lax.axis_index("s")
    wid = s * NC + c
    pltpu.sync_copy(tile_hbm.at[wid], tile_v)
    pltpu.sync_copy(lane_hbm.at[wid], lane_v)
    return wid


def _q_body(q3_hbm, tile_hbm, lane_hbm, rot_out,
            tile_v, lane_v, qb0, qb1, rot_v, sem):
    wid = _worker_id_and_idx(tile_hbm, lane_hbm, tile_v, lane_v)
    base = wid * BPW
    qbufs = (qb0, qb1)
    lanes = lax.iota(jnp.int32, 16)

    def fire(j, buf):
        return (pltpu.async_copy(q3_hbm.at[tile_v.at[j]], qbufs[buf], sem),)

    def extract(j, buf):
        qb = qbufs[buf]
        for g in range(CHUNK // 16):
            i0 = g * 16
            slot = i0 + lanes
            lane = plsc.load_gather(lane_v, [jnp.full((16,), j, jnp.int32),
                                             slot])
            row = (j * CHUNK + i0) + lanes

            def compq(col):
                return plsc.load_gather(
                    qb, [slot, jnp.full((16,), col, jnp.int32), lane])

            qr = compq(0)
            qi = compq(1)
            qj = compq(2)
            qk = compq(3)

            inv = 2.0 / (qr * qr + qi * qi + qj * qj + qk * qk)
            ii = qi * qi * inv
            jj = qj * qj * inv
            kk = qk * qk * inv
            ij = qi * qj * inv
            ik = qi * qk * inv
            jk = qj * qk * inv
            ir = qi * qr * inv
            jr = qj * qr * inv
            kr = qk * qr * inv

            def put(col, val):
                plsc.store_scatter(
                    rot_v, [row, jnp.full((16,), col, jnp.int32)], val)

            put(0, 1.0 - (jj + kk))
            put(1, ij - kr)
            put(2, ik + jr)
            put(3, ij + kr)
            put(4, 1.0 - (ii + kk))
            put(5, jk - ir)
            put(6, ik - jr)
            put(7, jk + ir)
            put(8, 1.0 - (ii + jj))

    cps = fire(0, 0)
    for j in range(NCH):
        for cp in cps:
            cp.wait()
        if j + 1 < NCH:
            cps = fire(j + 1, (j + 1) % 2)
        extract(j, j % 2)

    pltpu.sync_copy(rot_v, rot_out.at[pl.ds(base, BPW)])


def _t_body(t3_hbm, tile_hbm, lane_hbm, tran_out,
            tile_v, lane_v, tb0, tb1, tout_v, sem):
    wid = _worker_id_and_idx(tile_hbm, lane_hbm, tile_v, lane_v)
    base = wid * BPW
    tbufs = (tb0, tb1)
    lanes = lax.iota(jnp.int32, 16)

    def fire(j, buf):
        return (pltpu.async_copy(t3_hbm.at[tile_v.at[j]], tbufs[buf], sem),)

    def extract(j, buf):
        tb = tbufs[buf]
        for g in range(CHUNK // 16):
            i0 = g * 16
            slot = i0 + lanes
            lane = plsc.load_gather(lane_v, [jnp.full((16,), j, jnp.int32),
                                             slot])
            row = (j * CHUNK + i0) + lanes
            for col in range(2):
                cols = jnp.full((16,), col, jnp.int32)
                tval = plsc.load_gather(tb, [slot, cols, lane])
                plsc.store_scatter(tout_v, [row, cols], tval)

    cps = fire(0, 0)
    for j in range(NCH):
        for cp in cps:
            cp.wait()
        if j + 1 < NCH:
            cps = fire(j + 1, (j + 1) % 2)
        extract(j, j % 2)

    pltpu.sync_copy(tout_v, tran_out.at[pl.ds(base, BPW)])


_MESH = plsc.VectorSubcoreMesh(core_axis_name="c", subcore_axis_name="s")
_PARAMS = pltpu.CompilerParams(
    needs_layout_passes=False, use_tc_tiling_on_sc=False)

_q_call = functools.partial(
    pl.kernel,
    out_type=jax.ShapeDtypeStruct((B, 9), jnp.float32),
    mesh=_MESH,
    compiler_params=_PARAMS,
    scratch_types=[
        pltpu.VMEM((NCH, CHUNK), jnp.int32),
        pltpu.VMEM((NCH, CHUNK), jnp.int32),
        pltpu.VMEM((CHUNK, 4, 128), jnp.float32),
        pltpu.VMEM((CHUNK, 4, 128), jnp.float32),
        pltpu.VMEM((BPW, 9), jnp.float32),
        pltpu.SemaphoreType.DMA,
    ],
)(_q_body)

_t_call = functools.partial(
    pl.kernel,
    out_type=jax.ShapeDtypeStruct((B, 2), jnp.float32),
    mesh=_MESH,
    compiler_params=_PARAMS,
    scratch_types=[
        pltpu.VMEM((NCH, CHUNK), jnp.int32),
        pltpu.VMEM((NCH, CHUNK), jnp.int32),
        pltpu.VMEM((CHUNK, 2, 128), jnp.float32),
        pltpu.VMEM((CHUNK, 2, 128), jnp.float32),
        pltpu.VMEM((BPW, 2), jnp.float32),
        pltpu.SemaphoreType.DMA,
    ],
)(_t_body)


def _tile_view(table, ncomp):
    padded = jnp.pad(table, ((0, VPAD - N), (0, 0)))
    return padded.T.reshape(ncomp, NT, 128).transpose(1, 0, 2)


@jax.jit
def kernel(rots_emb_w, trans_emb_w, ind):
    ind32 = ind.astype(jnp.int32)
    tile = (ind32 >> 7).reshape(NW, NCH, CHUNK)
    lane = (ind32 & 127).reshape(NW, NCH, CHUNK)
    rot9 = _q_call(_tile_view(rots_emb_w, 4), tile, lane)
    tran = _t_call(_tile_view(trans_emb_w, 2), tile, lane)
    return rot9.reshape(B, 3, 3), tran
